# baseline (speedup 1.0000x reference)
import functools

import jax
import jax.numpy as jnp
from jax import lax
from jax.experimental import pallas as pl
from jax.experimental.pallas import tpu as pltpu
from jax.experimental.pallas import tpu_sc as plsc

_EMBED = 64
_OUT = 2
_SEQ = 200
_BATCH = 4096
_PAD = 8
_SEQP = _SEQ + _PAD
_HALF = _SEQP // 2
_REM = _SEQ - _HALF
_NC = 2
_NS = 16
_NW = _NC * _NS
_BPW = _BATCH // _NW


def _sc_body(textp_hbm, table_hbm, sums_hbm, idx_v, rows_v, out_v, sem):
    wid = lax.axis_index("s") * _NC + lax.axis_index("c")
    base = wid * _BPW
    pltpu.sync_copy(textp_hbm.at[pl.ds(base * 2, 2 * _BPW)], idx_v)

    def one(b, carry):
        h1 = pltpu.async_copy(
            table_hbm.at[idx_v.at[2 * b]],
            rows_v.at[pl.ds(0, _HALF)], sem)
        h2 = pltpu.async_copy(
            table_hbm.at[idx_v.at[2 * b + 1, pl.ds(0, _REM)]],
            rows_v.at[pl.ds(_HALF, _REM)], sem)
        h1.wait()
        h2.wait()
        zero = jnp.zeros((16,), jnp.float32)

        def srow(s, accs):
            return tuple(
                accs[c] + rows_v[s, pl.ds(c * 16, 16)] for c in range(4))

        accs = lax.fori_loop(0, _SEQ, srow, (zero,) * 4, unroll=8)
        for c in range(4):
            out_v[b, pl.ds(c * 16, 16)] = accs[c]
        return carry

    lax.fori_loop(0, _BPW, one, None)
    pltpu.sync_copy(out_v, sums_hbm.at[pl.ds(base, _BPW)])


def _sc_gather_sum(textp, table):
    f = pl.kernel(
        _sc_body,
        out_type=jax.ShapeDtypeStruct((_BATCH, _EMBED), jnp.float32),
        mesh=plsc.VectorSubcoreMesh(core_axis_name="c", subcore_axis_name="s"),
        compiler_params=pltpu.CompilerParams(use_tc_tiling_on_sc=False),
        scratch_types=[
            pltpu.VMEM((2 * _BPW, _HALF), jnp.int32),
            pltpu.VMEM((_SEQ, _EMBED), jnp.float32),
            pltpu.VMEM((_BPW, _EMBED), jnp.float32),
            pltpu.SemaphoreType.DMA,
        ],
    )
    return f(textp, table)


def _tc_body(sums_ref, text_ref, row0_ref, w_ref, b_ref, out_ref):
    cnt = jnp.sum((text_ref[...] == 0).astype(jnp.float32),
                  axis=1, keepdims=True) - float(_PAD)
    pooled = (sums_ref[...] - cnt * row0_ref[...]) * (1.0 / _SEQ)
    out_ref[...] = lax.dot_general(
        pooled, w_ref[...], (((1,), (1,)), ((), ())),
        preferred_element_type=jnp.float32,
        precision=lax.Precision.HIGHEST) + b_ref[...]


def _tc_finish(sums, textp2, row0, w, b2):
    return pl.pallas_call(
        _tc_body,
        out_shape=jax.ShapeDtypeStruct((_BATCH, _OUT), jnp.float32),
    )(sums, textp2, row0, w, b2)


def kernel(text, embed_weight, linear_W, linear_b):
    idx = text.astype(jnp.int32).T
    idxp = jnp.pad(idx, ((0, 0), (0, _PAD)))
    textp = idxp.reshape(2 * _BATCH, _HALF)
    sums = _sc_gather_sum(textp, embed_weight)
    row0 = embed_weight[0:1]
    out = _tc_finish(sums, idxp, row0, linear_W,
                     linear_b.reshape(1, _OUT))
    return out
